# baseline (speedup 1.0000x reference)
import functools

import jax
import jax.numpy as jnp
from jax import lax
from jax.experimental import pallas as pl
from jax.experimental.pallas import tpu as pltpu
from jax.experimental.pallas import tpu_sc as plsc

NUM_FIELDS = 26
DIM = 16
LANES = 16
IDX_W = 128
SUBS_PER_CHUNK = 4
CHUNK = IDX_W * SUBS_PER_CHUNK


def _make_gather(num_rows: int, vocab1: int, nw: int, nc: int):
    rows_w = num_rows // nw
    n_idx_rows = rows_w // IDX_W
    n_chunks = rows_w // CHUNK
    n_pairs = n_chunks // 2
    vregs_per_row = IDX_W // LANES

    mesh = plsc.VectorSubcoreMesh(core_axis_name="c", subcore_axis_name="s")

    @functools.partial(
        pl.kernel,
        out_type=jax.ShapeDtypeStruct((num_rows, DIM), jnp.float32),
        mesh=mesh,
        compiler_params=pltpu.CompilerParams(use_tc_tiling_on_sc=False),
        scratch_types=[
            pltpu.VMEM((n_idx_rows, IDX_W), jnp.int32),
            pltpu.VMEM((CHUNK, DIM), jnp.float32),
            pltpu.VMEM((CHUNK, DIM), jnp.float32),
            pltpu.SemaphoreType.DMA,
            pltpu.SemaphoreType.DMA,
            pltpu.SemaphoreType.DMA,
        ],
    )
    def gather_kernel(x_hbm, tab_hbm, out_hbm, g2d, rows0, rows1,
                      gsem, wsem0, wsem1):
        wid = lax.axis_index("s") * nc + lax.axis_index("c")
        row0 = wid * rows_w

        pltpu.sync_copy(x_hbm.at[pl.ds(wid * n_idx_rows, n_idx_rows)], g2d)

        def ixbody(r, fld):
            for k in range(vregs_per_row):
                sl = pl.ds(k * LANES, LANES)
                g2d[r, sl] = g2d[r, sl] + fld * vocab1
                t = fld + LANES
                fld = jnp.where(t >= NUM_FIELDS, t - NUM_FIELDS, t)
            return fld
        lax.fori_loop(0, n_idx_rows, ixbody, lax.iota(jnp.int32, LANES))

        def drain_wb(buf_ref, sem):
            pltpu.make_async_copy(out_hbm.at[pl.ds(0, CHUNK)], buf_ref,
                                  sem).wait()

        def do_chunk(c, buf_ref, sem):
            descs = []
            for j in range(SUBS_PER_CHUNK):
                idx_row = g2d.at[c * SUBS_PER_CHUNK + j]
                dst = buf_ref.at[pl.ds(j * IDX_W, IDX_W)]
                descs.append(pltpu.async_copy(tab_hbm.at[idx_row], dst, gsem))
            for d in descs:
                d.wait()
            pltpu.async_copy(buf_ref,
                             out_hbm.at[pl.ds(row0 + c * CHUNK, CHUNK)], sem)

        def pair_body(t, carry):
            @pl.when(t > 0)
            def _():
                drain_wb(rows0, wsem0)
            do_chunk(2 * t, rows0, wsem0)

            @pl.when(t > 0)
            def _():
                drain_wb(rows1, wsem1)
            do_chunk(2 * t + 1, rows1, wsem1)
            return carry
        lax.fori_loop(0, n_pairs, pair_body, 0)

        drain_wb(rows0, wsem0)
        drain_wb(rows1, wsem1)

    return gather_kernel


def kernel(x, tables):
    batch, num_fields = x.shape
    _, vocab1, dim = tables.shape
    num_rows = batch * num_fields

    info = plsc.get_sparse_core_info()
    nc, ns = info.num_cores, info.num_subcores
    nw = nc * ns

    assert dim == DIM and num_fields == NUM_FIELDS
    assert num_rows % (nw * CHUNK * 2) == 0
    assert (num_rows // nw) % NUM_FIELDS == 0

    tab_flat = tables.reshape(num_fields * vocab1, dim)
    x2d = x.astype(jnp.int32).reshape(num_rows // IDX_W, IDX_W)

    out = _make_gather(num_rows, vocab1, nw, nc)(x2d, tab_flat)
    return out.reshape(batch, num_fields * dim)


# speedup vs baseline: 31.4939x; 31.4939x over previous
import functools

import jax
import jax.numpy as jnp
from jax import lax
from jax.experimental import pallas as pl
from jax.experimental.pallas import tpu as pltpu
from jax.experimental.pallas import tpu_sc as plsc

NUM_FIELDS = 26
DIM = 16
LANES = 16
BCHUNK = 4096


def _make_lookup(batch: int, vocab: int, nw: int, nc: int):
    num_rows = NUM_FIELDS * DIM
    rows_per_tile = num_rows // nw
    n_chunks = batch // BCHUNK

    mesh = plsc.VectorSubcoreMesh(core_axis_name="c", subcore_axis_name="s")

    @functools.partial(
        pl.kernel,
        out_type=jax.ShapeDtypeStruct((num_rows, batch), jnp.float32),
        mesh=mesh,
        compiler_params=pltpu.CompilerParams(needs_layout_passes=False),
        scratch_types=[
            pltpu.VMEM((vocab,), jnp.float32),
            pltpu.VMEM((BCHUNK,), jnp.int32),
            pltpu.VMEM((BCHUNK,), jnp.float32),
        ],
    )
    def lookup_kernel(xt_hbm, tab_hbm, out_hbm, rowbuf, idxbuf, outbuf):
        wid = lax.axis_index("s") * nc + lax.axis_index("c")
        c0 = wid * rows_per_tile

        def row_body(r, _):
            c = c0 + r
            f = lax.shift_right_logical(c, 4)
            d = lax.bitwise_and(c, DIM - 1)
            pltpu.sync_copy(tab_hbm.at[f, d], rowbuf)

            def chunk_body(k, _):
                b0 = k * BCHUNK
                pltpu.sync_copy(xt_hbm.at[f, pl.ds(b0, BCHUNK)], idxbuf)

                def gbody(j, _):
                    sl = pl.ds(j * LANES, LANES)
                    vals = plsc.load_gather(rowbuf, [idxbuf[sl]])
                    outbuf[sl] = vals
                    return 0
                lax.fori_loop(0, BCHUNK // LANES, gbody, 0)
                pltpu.sync_copy(outbuf, out_hbm.at[c, pl.ds(b0, BCHUNK)])
                return 0
            lax.fori_loop(0, n_chunks, chunk_body, 0)
            return 0
        lax.fori_loop(0, rows_per_tile, row_body, 0)

    return lookup_kernel


def kernel(x, tables):
    batch, num_fields = x.shape
    _, vocab, dim = tables.shape

    info = plsc.get_sparse_core_info()
    nw = info.num_cores * info.num_subcores

    assert dim == DIM and num_fields == NUM_FIELDS
    assert (num_fields * dim) % nw == 0 and batch % BCHUNK == 0

    xt = jnp.swapaxes(x.astype(jnp.int32), 0, 1)
    tab_t = jnp.swapaxes(tables, 1, 2)

    out_t = _make_lookup(batch, vocab, nw, info.num_cores)(xt, tab_t)
    return jnp.swapaxes(out_t, 0, 1).reshape(batch, num_fields * dim)


# speedup vs baseline: 56.5948x; 1.7970x over previous
import functools

import jax
import jax.numpy as jnp
from jax import lax
from jax.experimental import pallas as pl
from jax.experimental.pallas import tpu as pltpu
from jax.experimental.pallas import tpu_sc as plsc

NUM_FIELDS = 26
DIM = 16
LANES = 16
BCHUNK = 4096


def _make_lookup(batch: int, vocab: int, nw: int, nc: int):
    num_rows = NUM_FIELDS * DIM
    rows_per_tile = num_rows // nw
    n_chunks = batch // BCHUNK

    mesh = plsc.VectorSubcoreMesh(core_axis_name="c", subcore_axis_name="s")

    @functools.partial(
        pl.kernel,
        out_type=jax.ShapeDtypeStruct((num_rows, batch), jnp.float32),
        mesh=mesh,
        compiler_params=pltpu.CompilerParams(needs_layout_passes=False),
        scratch_types=[
            pltpu.VMEM((vocab,), jnp.float32),
            pltpu.VMEM((batch,), jnp.int32),
            pltpu.VMEM((BCHUNK,), jnp.float32),
            pltpu.VMEM((BCHUNK,), jnp.float32),
            pltpu.SemaphoreType.DMA,
        ],
    )
    def lookup_kernel(xt_hbm, tab_hbm, out_hbm, rowbuf, idxbuf,
                      outbuf0, outbuf1, wsem):
        wid = lax.axis_index("s") * nc + lax.axis_index("c")
        c0 = wid * rows_per_tile
        outbufs = [outbuf0, outbuf1]

        def drain_wb():
            pltpu.make_async_copy(out_hbm.at[0, pl.ds(0, BCHUNK)],
                                  outbuf0, wsem).wait()

        def row_body(r, _):
            c = c0 + r
            f = lax.shift_right_logical(c, 4)
            d = lax.bitwise_and(c, DIM - 1)
            pltpu.sync_copy(tab_hbm.at[f, d], rowbuf)
            pltpu.sync_copy(xt_hbm.at[f], idxbuf)

            wbs = {}
            for k in range(n_chunks):
                buf = outbufs[k % 2]
                if k >= 2:
                    wbs[k - 2].wait()
                else:
                    @pl.when(r > 0)
                    def _():
                        drain_wb()

                @plsc.parallel_loop(0, BCHUNK // LANES, unroll=8)
                def _(j):
                    vals = plsc.load_gather(
                        rowbuf, [idxbuf[pl.ds(k * BCHUNK + j * LANES, LANES)]])
                    buf[pl.ds(j * LANES, LANES)] = vals

                wbs[k] = pltpu.async_copy(
                    buf, out_hbm.at[c, pl.ds(k * BCHUNK, BCHUNK)], wsem)
            return 0
        lax.fori_loop(0, rows_per_tile, row_body, 0)
        drain_wb()
        drain_wb()

    return lookup_kernel


def kernel(x, tables):
    batch, num_fields = x.shape
    _, vocab, dim = tables.shape

    info = plsc.get_sparse_core_info()
    nw = info.num_cores * info.num_subcores

    assert dim == DIM and num_fields == NUM_FIELDS
    assert (num_fields * dim) % nw == 0 and batch % BCHUNK == 0

    xt = jnp.swapaxes(x.astype(jnp.int32), 0, 1)
    tab_t = jnp.swapaxes(tables, 1, 2)

    out_t = _make_lookup(batch, vocab, nw, info.num_cores)(xt, tab_t)
    return jnp.swapaxes(out_t, 0, 1).reshape(batch, num_fields * dim)


# speedup vs baseline: 65.2721x; 1.1533x over previous
import functools

import jax
import jax.numpy as jnp
from jax import lax
from jax.experimental import pallas as pl
from jax.experimental.pallas import tpu as pltpu
from jax.experimental.pallas import tpu_sc as plsc

NUM_FIELDS = 26
DIM = 16
LANES = 16
BCHUNK = 4096


def _make_lookup(batch: int, vocab: int, nw: int, nc: int):
    num_rows = NUM_FIELDS * DIM
    rows_per_tile = num_rows // nw
    n_chunks = batch // BCHUNK

    mesh = plsc.VectorSubcoreMesh(core_axis_name="c", subcore_axis_name="s")

    @functools.partial(
        pl.kernel,
        out_type=jax.ShapeDtypeStruct((num_rows, batch), jnp.float32),
        mesh=mesh,
        compiler_params=pltpu.CompilerParams(needs_layout_passes=False),
        scratch_types=[
            pltpu.VMEM((vocab,), jnp.float32),
            pltpu.VMEM((batch,), jnp.int32),
            pltpu.VMEM((BCHUNK,), jnp.float32),
            pltpu.VMEM((BCHUNK,), jnp.float32),
            pltpu.SemaphoreType.DMA,
        ],
    )
    def lookup_kernel(xt_hbm, tab_hbm, out_hbm, rowbuf, idxbuf,
                      outbuf0, outbuf1, wsem):
        wid = lax.axis_index("s") * nc + lax.axis_index("c")
        c0 = wid * rows_per_tile
        outbufs = [outbuf0, outbuf1]

        def drain_wb():
            pltpu.make_async_copy(out_hbm.at[0, pl.ds(0, BCHUNK)],
                                  outbuf0, wsem).wait()

        def row_body(r, fprev):
            c = c0 + r
            f = lax.shift_right_logical(c, 4)
            d = lax.bitwise_and(c, DIM - 1)
            pltpu.sync_copy(tab_hbm.at[f, d], rowbuf)

            @pl.when(f != fprev)
            def _():
                pltpu.sync_copy(xt_hbm.at[f], idxbuf)

            wbs = {}
            for k in range(n_chunks):
                buf = outbufs[k % 2]
                if k >= 2:
                    wbs[k - 2].wait()
                else:
                    @pl.when(r > 0)
                    def _():
                        drain_wb()

                @plsc.parallel_loop(0, BCHUNK // LANES, unroll=8)
                def _(j):
                    vals = plsc.load_gather(
                        rowbuf, [idxbuf[pl.ds(k * BCHUNK + j * LANES, LANES)]])
                    buf[pl.ds(j * LANES, LANES)] = vals

                wbs[k] = pltpu.async_copy(
                    buf, out_hbm.at[c, pl.ds(k * BCHUNK, BCHUNK)], wsem)
            return f
        lax.fori_loop(0, rows_per_tile, row_body, jnp.int32(-1))
        drain_wb()
        drain_wb()

    return lookup_kernel


def kernel(x, tables):
    batch, num_fields = x.shape
    _, vocab, dim = tables.shape

    info = plsc.get_sparse_core_info()
    nw = info.num_cores * info.num_subcores

    assert dim == DIM and num_fields == NUM_FIELDS
    assert (num_fields * dim) % nw == 0 and batch % BCHUNK == 0

    xt = jnp.swapaxes(x.astype(jnp.int32), 0, 1)
    tab_t = jnp.swapaxes(tables, 1, 2)

    out_t = _make_lookup(batch, vocab, nw, info.num_cores)(xt, tab_t)
    return jnp.swapaxes(out_t, 0, 1).reshape(batch, num_fields * dim)
